# baseline (speedup 1.0000x reference)
import functools

import jax
import jax.numpy as jnp
from jax import lax
from jax.experimental import pallas as pl
from jax.experimental.pallas import tpu as pltpu
from jax.experimental.pallas import tpu_sc as plsc

C_DIM = 128
DIM = 3
HID = 128
NB = 6
EK = 128
K = 20
B, T = 2, 4096
CTX = 1 + 2 * DIM
R = B * T

ROWS_BLK = 2048

CHUNK = 1024
NCHUNK = T // CHUNK
CPS = HID // 16


def _wpool_body(tab_hbm, idx_hbm, w_hbm, out_hbm, tab_v, idx_v, w_v, out_v):
    core = lax.axis_index("c")
    sub = lax.axis_index("s")
    roff = sub * CPS
    coff = core * T
    for c in range(CPS):
        pltpu.sync_copy(tab_hbm.at[roff + c, pl.ds(coff, T)],
                        tab_v.at[pl.ds(c * T, T)])

    @pl.loop(0, NCHUNK)
    def _chunk(ch):
        base = ch * CHUNK
        pltpu.sync_copy(idx_hbm.at[pl.ds((coff + base) * 32, CHUNK * 32)], idx_v)
        pltpu.sync_copy(w_hbm.at[pl.ds((coff + base) * K, CHUNK * K)], w_v)

        @pl.loop(0, CHUNK // 16)
        def _grp(g):
            goff = g * 16
            wbase = _iota16() * K + goff * K
            ibase = _iota16() * 32 + goff * 32
            accs = [None] * CPS
            for k in range(K):
                ir = plsc.load_gather(idx_v, [ibase + k])
                wv = plsc.load_gather(w_v, [wbase + k])
                for c in range(CPS):
                    val = plsc.load_gather(tab_v, [ir + (c * T)])
                    accs[c] = wv * val if k == 0 else accs[c] + wv * val
            for c in range(CPS):
                out_v[c, pl.ds(goff, 16)] = accs[c]

        pltpu.sync_copy(out_v, out_hbm.at[pl.ds(roff, CPS),
                                          pl.ds(coff + base, CHUNK)])


@functools.partial(
    pl.kernel,
    mesh=plsc.VectorSubcoreMesh(core_axis_name="c", subcore_axis_name="s"),
    out_type=jax.ShapeDtypeStruct((HID, R), jnp.float32),
    compiler_params=pltpu.CompilerParams(needs_layout_passes=False),
    scratch_types=[
        pltpu.VMEM((CPS * T,), jnp.float32),
        pltpu.VMEM((CHUNK * 32,), jnp.int32),
        pltpu.VMEM((CHUNK * K,), jnp.float32),
        pltpu.VMEM((CPS, CHUNK), jnp.float32),
    ],
)
def _wpool_sc(tab_hbm, idx_hbm, w_hbm, out_hbm, *scratch):
    _wpool_body(tab_hbm, idx_hbm, w_hbm, out_hbm, *scratch)


RT = 256
NCM = 128
CAND = 1024
ROWS_SEL = R // 32


def _d2_body(p_ref, pT_ref, d2_ref, cm_ref):
    pt = p_ref[0]
    pTf = pT_ref[0]
    sqr = jnp.sum(pt * pt, axis=1, keepdims=True)
    sqc = jnp.sum(pTf * pTf, axis=0, keepdims=True)
    d2 = jnp.maximum(
        sqr + sqc - 2.0 * jnp.dot(pt, pTf, preferred_element_type=jnp.float32),
        0.0)
    d2_ref[0] = d2
    cm = d2[:, 0:NCM]
    for m in range(1, T // NCM):
        cm = jnp.minimum(cm, d2[:, m * NCM:(m + 1) * NCM])
    cm_ref[...] = cm


def _d2_call(p, pT):
    grid = (B * (T // RT),)
    return pl.pallas_call(
        _d2_body,
        grid=grid,
        in_specs=[pl.BlockSpec((1, RT, DIM), lambda i: (i // (T // RT), i % (T // RT), 0)),
                  pl.BlockSpec((1, DIM, T), lambda i: (i // (T // RT), 0, 0))],
        out_specs=[pl.BlockSpec((1, RT, T), lambda i: (i // (T // RT), i % (T // RT), 0)),
                   pl.BlockSpec((RT, NCM), lambda i: (i, 0))],
        out_shape=[jax.ShapeDtypeStruct((B, T, T), jnp.float32),
                   jax.ShapeDtypeStruct((R, NCM), jnp.float32)],
    )(p, pT)


def _iota16():
    return lax.iota(jnp.int32, 16)


_INF = float('inf')


def _merge16(a, b):
    rb = lax.rev(b, (0,))
    lo = lax.sort(jnp.minimum(a, rb))
    hi = lax.sort(jnp.maximum(a, rb))
    return lo, hi


def _merge32_low(a0, a1, b0, b1):
    x0 = jnp.minimum(a0, lax.rev(b1, (0,)))
    x1 = jnp.minimum(a1, lax.rev(b0, (0,)))
    y0 = lax.sort(jnp.minimum(x0, x1))
    y1 = lax.sort(jnp.maximum(x0, x1))
    return y0, y1


def _kv_minmax(ka, va, kb, vb):
    m = ka <= kb
    return (jnp.where(m, ka, kb), jnp.where(m, va, vb),
            jnp.where(m, kb, ka), jnp.where(m, vb, va))


def _kv_merge32_low(a0, av0, a1, av1, b0, bv0, b1, bv1):
    rb0, rbv0 = lax.rev(b1, (0,)), lax.rev(bv1, (0,))
    rb1, rbv1 = lax.rev(b0, (0,)), lax.rev(bv0, (0,))
    x0, xv0, _, _ = _kv_minmax(a0, av0, rb0, rbv0)
    x1, xv1, _, _ = _kv_minmax(a1, av1, rb1, rbv1)
    y0, yv0, y1, yv1 = _kv_minmax(x0, xv0, x1, xv1)
    y0, yv0 = plsc.sort_key_val(y0, yv0)
    y1, yv1 = plsc.sort_key_val(y1, yv1)
    return y0, yv0, y1, yv1


def _sel_body(d2_hbm, cm_hbm, pT_hbm, val_hbm, idx_hbm, pool_hbm,
              cm_v, row_v, cval_v, cidx_v, vout_v, iout_v, ptab_v, pout_v,
              sem0, sem1):
    wid = lax.axis_index("s") * 2 + lax.axis_index("c")
    rbase = wid * ROWS_SEL
    bb = rbase // T
    pltpu.sync_copy(cm_hbm.at[pl.ds(rbase, ROWS_SEL)], cm_v)
    for d in range(DIM):
        pltpu.sync_copy(pT_hbm.at[pl.ds((bb * DIM + d) * T, T)],
                        ptab_v.at[pl.ds(d * T, T)])
    sems = [sem0, sem1]

    def _issue(r, b):
        pltpu.async_copy(d2_hbm.at[bb, (rbase % T) + r, pl.ds(0, T)],
                         row_v.at[b], sems[b])

    for b in range(2):
        _issue(b, b)

    def _row(r, b):
        pltpu.make_async_copy(d2_hbm.at[bb, 0, pl.ds(0, T)],
                              row_v.at[b], sems[b]).wait()
        s = [lax.sort(cm_v[r, pl.ds(i * 16, 16)]) for i in range(8)]
        m = [_merge16(s[2 * i], s[2 * i + 1]) for i in range(4)]
        n0 = _merge32_low(*m[0], *m[1])
        n1 = _merge32_low(*m[2], *m[3])
        f0, f1 = _merge32_low(*n0, *n1)
        tau = jnp.full((16,), f1[3])

        QT = T // 4
        QCAP = 272

        def _scan(i, offs):
            new = []
            for q in range(4):
                off = offs[q]
                v = row_v[b, pl.ds(q * QT + i * 16, 16)]
                msk = v <= tau
                plsc.store_compressed(cval_v.at[pl.ds(off, 16)], v, mask=msk)
                plsc.store_compressed(cidx_v.at[pl.ds(off, 16)],
                                      _iota16() + (q * QT + i * 16), mask=msk)
                new.append(off + plsc.all_reduce_population_count(msk)[0])
            return tuple(new)

        offs = lax.fori_loop(0, QT // 16, _scan,
                             tuple(jnp.int32(q * QCAP) for q in range(4)),
                             unroll=2)
        for q in range(4):
            cval_v[pl.ds(offs[q], 16)] = jnp.full((16,), _INF)

        def _sel(i, carry):
            k0, v0, k1, v1 = carry
            ck = cval_v[pl.ds(i * 16, 16)]
            ci = cidx_v[pl.ds(i * 16, 16)]
            ck, ci = plsc.sort_key_val(ck, ci)
            return _kv_merge32_low(k0, v0, k1, v1,
                                   ck, ci, jnp.full((16,), _INF), ci)

        carry = (jnp.full((16,), _INF), _iota16(),
                 jnp.full((16,), _INF), _iota16())
        for q in range(4):
            carry = lax.fori_loop(q * (QCAP // 16), (offs[q] >> 4) + 1,
                                  _sel, carry)
        k0, v0, k1, v1 = carry
        vout_v[pl.ds(r * 32, 16)] = k0
        vout_v[pl.ds(r * 32 + 16, 16)] = k1
        iout_v[pl.ds(r * 32, 16)] = v0
        iout_v[pl.ds(r * 32 + 16, 16)] = v1
        for d in range(DIM):
            pout_v[pl.ds((d * ROWS_SEL + r) * 32, 16)] = \
                plsc.load_gather(ptab_v, [v0 + d * T])
            pout_v[pl.ds((d * ROWS_SEL + r) * 32 + 16, 16)] = \
                plsc.load_gather(ptab_v, [v1 + d * T])
        nxt = r + 2

        @pl.when(nxt < ROWS_SEL)
        def _():
            _issue(nxt, b)

    @pl.loop(0, ROWS_SEL, step=2)
    def _rows(r):
        for b in range(2):
            _row(r + b, b)

    pltpu.sync_copy(vout_v, val_hbm.at[pl.ds(rbase * 32, ROWS_SEL * 32)])
    pltpu.sync_copy(iout_v, idx_hbm.at[pl.ds(rbase * 32, ROWS_SEL * 32)])
    for d in range(DIM):
        pltpu.sync_copy(pout_v.at[pl.ds(d * ROWS_SEL * 32, ROWS_SEL * 32)],
                        pool_hbm.at[pl.ds((d * R + rbase) * 32, ROWS_SEL * 32)])


@functools.partial(
    pl.kernel,
    mesh=plsc.VectorSubcoreMesh(core_axis_name="c", subcore_axis_name="s"),
    out_type=[jax.ShapeDtypeStruct((R * 32,), jnp.float32),
              jax.ShapeDtypeStruct((R * 32,), jnp.int32),
              jax.ShapeDtypeStruct((DIM * R * 32,), jnp.float32)],
    compiler_params=pltpu.CompilerParams(needs_layout_passes=False),
    scratch_types=[
        pltpu.VMEM((ROWS_SEL, NCM), jnp.float32),
        pltpu.VMEM((2, T), jnp.float32),
        pltpu.VMEM((CAND + 128,), jnp.float32),
        pltpu.VMEM((CAND + 128,), jnp.int32),
        pltpu.VMEM((ROWS_SEL * 32,), jnp.float32),
        pltpu.VMEM((ROWS_SEL * 32,), jnp.int32),
        pltpu.VMEM((DIM * T,), jnp.float32),
        pltpu.VMEM((DIM * ROWS_SEL * 32,), jnp.float32),
        pltpu.SemaphoreType.DMA,
        pltpu.SemaphoreType.DMA,
    ],
)
def _sel_sc(d2_hbm, cm_hbm, pT_hbm, val_hbm, idx_hbm, pool_hbm, *scratch):
    _sel_body(d2_hbm, cm_hbm, pT_hbm, val_hbm, idx_hbm, pool_hbm, *scratch)


SROWS = 128


def _scores_body(ctxT_ref, ws1T_ref, bs1_ref, ws2_ref, s_ref):
    ctx_t = ctxT_ref[...]
    for i in range(NB):
        Et = jnp.dot(ws1T_ref[i], ctx_t, preferred_element_type=jnp.float32)
        Et = jnp.maximum(Et + bs1_ref[i][:, None], 0.0)
        s_ref[i] = jnp.dot(ws2_ref[i][None, :], Et,
                           preferred_element_type=jnp.float32)[0]


def _scores_call(ctxT, ws1T, bs1, ws2):
    grid = (R // SROWS,)
    return pl.pallas_call(
        _scores_body,
        grid=grid,
        in_specs=[pl.BlockSpec((CTX, SROWS * K), lambda i: (0, i)),
                  pl.BlockSpec((NB, EK, CTX), lambda i: (0, 0, 0)),
                  pl.BlockSpec((NB, EK), lambda i: (0, 0)),
                  pl.BlockSpec((NB, EK), lambda i: (0, 0))],
        out_specs=pl.BlockSpec((NB, SROWS * K), lambda i: (0, i)),
        out_shape=jax.ShapeDtypeStruct((NB, R * K), jnp.float32),
    )(ctxT, ws1T, bs1, ws2)


def _softmax_body(s_ref, a_ref):
    sm = s_ref[0]
    sm = sm - jnp.max(sm, axis=1, keepdims=True)
    ex = jnp.exp(sm)
    a_ref[0] = ex / jnp.sum(ex, axis=1, keepdims=True)


def _softmax_call(s3):
    grid = (NB * (R // 1024),)
    nrt = R // 1024
    return pl.pallas_call(
        _softmax_body,
        grid=grid,
        in_specs=[pl.BlockSpec((1, 1024, K), lambda i: (i // nrt, i % nrt, 0))],
        out_specs=pl.BlockSpec((1, 1024, K), lambda i: (i // nrt, i % nrt, 0)),
        out_shape=jax.ShapeDtypeStruct((NB, R, K), jnp.float32),
    )(s3)


def _affine_body(wt_ref, b_ref, x_ref, out_ref):
    out_ref[...] = (jnp.dot(wt_ref[...], x_ref[...],
                            preferred_element_type=jnp.float32) + b_ref[...])


def _affine_call(wt, b, x):
    din, dout = wt.shape[1], wt.shape[0]
    grid = (R // ROWS_BLK,)
    return pl.pallas_call(
        _affine_body,
        grid=grid,
        in_specs=[pl.BlockSpec((dout, din), lambda i: (0, 0)),
                  pl.BlockSpec((dout, 1), lambda i: (0, 0)),
                  pl.BlockSpec((din, ROWS_BLK), lambda i: (0, i))],
        out_specs=pl.BlockSpec((dout, ROWS_BLK), lambda i: (0, i)),
        out_shape=jax.ShapeDtypeStruct((dout, R), jnp.float32),
    )(wt, b.reshape(dout, 1), x)


def _block_body(net_ref, wpool_ref, last_ref,
                WvT_ref, bv_ref, fc0aT_ref, fc0bT_ref, fc0bias_ref,
                fc1T_ref, fc1b_ref, scaT_ref, scbT_ref,
                out_ref):
    net = net_ref[...]
    att = jnp.dot(WvT_ref[...], wpool_ref[...],
                  preferred_element_type=jnp.float32) + bv_ref[...]
    rnet = jnp.maximum(net, 0.0)
    ratt = jnp.maximum(att, 0.0)
    h = (jnp.dot(fc0aT_ref[...], rnet, preferred_element_type=jnp.float32)
         + jnp.dot(fc0bT_ref[...], ratt, preferred_element_type=jnp.float32)
         + fc0bias_ref[...])
    dx = jnp.dot(fc1T_ref[...], jnp.maximum(h, 0.0),
                 preferred_element_type=jnp.float32) + fc1b_ref[...]
    sc = (jnp.dot(scaT_ref[...], net, preferred_element_type=jnp.float32)
          + jnp.dot(scbT_ref[...], att, preferred_element_type=jnp.float32))
    out_ref[...] = sc + dx + last_ref[...]


def _run_block(netT, wpoolT, lastT, WvT, bv, fc0T, fc0bias, fc1T, fc1b, scT):
    grid = (R // ROWS_BLK,)
    row_spec = pl.BlockSpec((HID, ROWS_BLK), lambda i: (0, i))
    w_spec = pl.BlockSpec((HID, HID), lambda i: (0, 0))
    b_spec = pl.BlockSpec((HID, 1), lambda i: (0, 0))
    return pl.pallas_call(
        _block_body,
        grid=grid,
        in_specs=[row_spec, row_spec, row_spec,
                  w_spec, b_spec, w_spec, w_spec, b_spec,
                  w_spec, b_spec, w_spec, w_spec],
        out_specs=row_spec,
        out_shape=jax.ShapeDtypeStruct((HID, R), jnp.float32),
    )(netT, wpoolT, lastT,
      WvT, bv.reshape(HID, 1), fc0T[:, :HID], fc0T[:, HID:],
      fc0bias.reshape(HID, 1), fc1T, fc1b.reshape(HID, 1),
      scT[:, :HID], scT[:, HID:])


def kernel(p, fc_pos_W, fc_pos_b, fc0_W, fc0_b, fc1_W, fc1_b, sc_W,
           att_Ws1, att_bs1, att_Ws2, att_bs2, att_Wv, att_bv, fcc_W, fcc_b):
    pT = jnp.transpose(p, (0, 2, 1))
    d2full, cmins = _d2_call(p, pT)
    selval, selidx, poolT = _sel_sc(d2full, cmins, pT.reshape(B * DIM * T))
    selval = selval.reshape(R, 32)
    selidx = selidx.reshape(R, 32)
    poolT = poolT.reshape(DIM, R, 32)
    idx = selidx[:, :K].reshape(B, T, K)
    dis = jnp.sqrt(jnp.maximum(selval[:, :K], 1e-12)).reshape(B, T, K)

    prep = jnp.broadcast_to(p.reshape(R, DIM).T[:, :, None], (DIM, R, K))
    ctxT = jnp.concatenate(
        [dis.reshape(1, R * K), poolT[:, :, :K].reshape(DIM, R * K),
         prep.reshape(DIM, R * K)], axis=0)
    s = _scores_call(ctxT, jnp.transpose(att_Ws1, (0, 2, 1)),
                     att_bs1, att_Ws2[:, :, 0])
    a3 = _softmax_call(s.reshape(NB, R, K))
    wflat = a3.reshape(NB, R * K)

    idxflat = selidx.reshape(R * 32)

    netT = _affine_call(fc_pos_W.T, fc_pos_b, p.reshape(R, DIM).T)
    lastT = jnp.zeros_like(netT)
    for i in range(NB):
        wpoolT = _wpool_sc(netT, idxflat, wflat[i])
        netT = _run_block(netT, wpoolT, lastT,
                          att_Wv[i].T, att_bv[i],
                          fc0_W[i].T, fc0_b[i], fc1_W[i].T, fc1_b[i],
                          sc_W[i].T)
        lastT = netT
    cT = _affine_call(fcc_W.T, fcc_b, netT)
    return cT.T.reshape(B, T, C_DIM)


# speedup vs baseline: 1.1719x; 1.1719x over previous
import functools

import jax
import jax.numpy as jnp
from jax import lax
from jax.experimental import pallas as pl
from jax.experimental.pallas import tpu as pltpu
from jax.experimental.pallas import tpu_sc as plsc

C_DIM = 128
DIM = 3
HID = 128
NB = 6
EK = 128
K = 20
B, T = 2, 4096
CTX = 1 + 2 * DIM
R = B * T

ROWS_BLK = 2048

CHUNK = 1024
NCHUNK = T // CHUNK
CPS = HID // 16


def _wpool_body(tab_hbm, idx_hbm, w_hbm, out_hbm, tab_v, idx_v, w_v, out_v):
    core = lax.axis_index("c")
    sub = lax.axis_index("s")
    roff = sub * CPS
    coff = core * T
    for c in range(CPS):
        pltpu.sync_copy(tab_hbm.at[roff + c, pl.ds(coff, T)],
                        tab_v.at[pl.ds(c * T, T)])

    @pl.loop(0, NCHUNK)
    def _chunk(ch):
        base = ch * CHUNK
        pltpu.sync_copy(idx_hbm.at[:, pl.ds(coff + base, CHUNK)], idx_v)
        pltpu.sync_copy(w_hbm.at[pl.ds((coff + base) * K, CHUNK * K)], w_v)

        @pl.loop(0, CHUNK // 16)
        def _grp(g):
            goff = g * 16
            wbase = _iota16() * K + goff * K
            accs = [None] * CPS
            for k in range(K):
                ir = idx_v[k, pl.ds(goff, 16)]
                wv = plsc.load_gather(w_v, [wbase + k])
                for c in range(CPS):
                    val = plsc.load_gather(tab_v, [ir + (c * T)])
                    accs[c] = wv * val if k == 0 else accs[c] + wv * val
            for c in range(CPS):
                out_v[c, pl.ds(goff, 16)] = accs[c]

        pltpu.sync_copy(out_v, out_hbm.at[pl.ds(roff, CPS),
                                          pl.ds(coff + base, CHUNK)])


@functools.partial(
    pl.kernel,
    mesh=plsc.VectorSubcoreMesh(core_axis_name="c", subcore_axis_name="s"),
    out_type=jax.ShapeDtypeStruct((HID, R), jnp.float32),
    compiler_params=pltpu.CompilerParams(needs_layout_passes=False),
    scratch_types=[
        pltpu.VMEM((CPS * T,), jnp.float32),
        pltpu.VMEM((K, CHUNK), jnp.int32),
        pltpu.VMEM((CHUNK * K,), jnp.float32),
        pltpu.VMEM((CPS, CHUNK), jnp.float32),
    ],
)
def _wpool_sc(tab_hbm, idx_hbm, w_hbm, out_hbm, *scratch):
    _wpool_body(tab_hbm, idx_hbm, w_hbm, out_hbm, *scratch)


RT = 256
NCM = 128
CAND = 1024
ROWS_SEL = R // 32


def _d2_body(p_ref, pT_ref, d2_ref, cm_ref):
    pt = p_ref[0]
    pTf = pT_ref[0]
    sqr = jnp.sum(pt * pt, axis=1, keepdims=True)
    sqc = jnp.sum(pTf * pTf, axis=0, keepdims=True)
    d2 = jnp.maximum(
        sqr + sqc - 2.0 * jnp.dot(pt, pTf, preferred_element_type=jnp.float32),
        0.0)
    d2_ref[0] = d2
    cm = d2[:, 0:NCM]
    for m in range(1, T // NCM):
        cm = jnp.minimum(cm, d2[:, m * NCM:(m + 1) * NCM])
    cm_ref[...] = cm


def _d2_call(p, pT):
    grid = (B * (T // RT),)
    return pl.pallas_call(
        _d2_body,
        grid=grid,
        in_specs=[pl.BlockSpec((1, RT, DIM), lambda i: (i // (T // RT), i % (T // RT), 0)),
                  pl.BlockSpec((1, DIM, T), lambda i: (i // (T // RT), 0, 0))],
        out_specs=[pl.BlockSpec((1, RT, T), lambda i: (i // (T // RT), i % (T // RT), 0)),
                   pl.BlockSpec((RT, NCM), lambda i: (i, 0))],
        out_shape=[jax.ShapeDtypeStruct((B, T, T), jnp.float32),
                   jax.ShapeDtypeStruct((R, NCM), jnp.float32)],
    )(p, pT)


def _iota16():
    return lax.iota(jnp.int32, 16)


_INF = float('inf')


def _merge16(a, b):
    rb = lax.rev(b, (0,))
    lo = lax.sort(jnp.minimum(a, rb))
    hi = lax.sort(jnp.maximum(a, rb))
    return lo, hi


def _merge32_low(a0, a1, b0, b1):
    x0 = jnp.minimum(a0, lax.rev(b1, (0,)))
    x1 = jnp.minimum(a1, lax.rev(b0, (0,)))
    y0 = lax.sort(jnp.minimum(x0, x1))
    y1 = lax.sort(jnp.maximum(x0, x1))
    return y0, y1


def _kv_minmax(ka, va, kb, vb):
    m = ka <= kb
    return (jnp.where(m, ka, kb), jnp.where(m, va, vb),
            jnp.where(m, kb, ka), jnp.where(m, vb, va))


def _kv_merge32_low(a0, av0, a1, av1, b0, bv0, b1, bv1):
    rb0, rbv0 = lax.rev(b1, (0,)), lax.rev(bv1, (0,))
    rb1, rbv1 = lax.rev(b0, (0,)), lax.rev(bv0, (0,))
    x0, xv0, _, _ = _kv_minmax(a0, av0, rb0, rbv0)
    x1, xv1, _, _ = _kv_minmax(a1, av1, rb1, rbv1)
    y0, yv0, y1, yv1 = _kv_minmax(x0, xv0, x1, xv1)
    y0, yv0 = plsc.sort_key_val(y0, yv0)
    y1, yv1 = plsc.sort_key_val(y1, yv1)
    return y0, yv0, y1, yv1


def _sel_body(d2_hbm, cm_hbm, pT_hbm, val_hbm, idx_hbm, pool_hbm,
              cm_v, row_v, cval_v, cidx_v, vout_v, iout_v, ptab_v, pout_v,
              sem0, sem1):
    wid = lax.axis_index("s") * 2 + lax.axis_index("c")
    rbase = wid * ROWS_SEL
    bb = rbase // T
    pltpu.sync_copy(cm_hbm.at[pl.ds(rbase, ROWS_SEL)], cm_v)
    for d in range(DIM):
        pltpu.sync_copy(pT_hbm.at[pl.ds((bb * DIM + d) * T, T)],
                        ptab_v.at[pl.ds(d * T, T)])
    sems = [sem0, sem1]

    def _issue(r, b):
        pltpu.async_copy(d2_hbm.at[bb, (rbase % T) + r, pl.ds(0, T)],
                         row_v.at[b], sems[b])

    for b in range(2):
        _issue(b, b)

    def _row(r, b):
        pltpu.make_async_copy(d2_hbm.at[bb, 0, pl.ds(0, T)],
                              row_v.at[b], sems[b]).wait()
        s = [lax.sort(cm_v[r, pl.ds(i * 16, 16)]) for i in range(8)]
        m = [_merge16(s[2 * i], s[2 * i + 1]) for i in range(4)]
        n0 = _merge32_low(*m[0], *m[1])
        n1 = _merge32_low(*m[2], *m[3])
        f0, f1 = _merge32_low(*n0, *n1)
        tau = jnp.full((16,), f1[3])

        QT = T // 4
        QCAP = 272

        def _scan(i, offs):
            new = []
            for q in range(4):
                off = offs[q]
                v = row_v[b, pl.ds(q * QT + i * 16, 16)]
                msk = v <= tau
                plsc.store_compressed(cval_v.at[pl.ds(off, 16)], v, mask=msk)
                plsc.store_compressed(cidx_v.at[pl.ds(off, 16)],
                                      _iota16() + (q * QT + i * 16), mask=msk)
                new.append(off + plsc.all_reduce_population_count(msk)[0])
            return tuple(new)

        offs = lax.fori_loop(0, QT // 16, _scan,
                             tuple(jnp.int32(q * QCAP) for q in range(4)),
                             unroll=2)
        for q in range(4):
            cval_v[pl.ds(offs[q], 16)] = jnp.full((16,), _INF)

        def _sel(i, carry):
            k0, v0, k1, v1 = carry
            ck = cval_v[pl.ds(i * 16, 16)]
            ci = cidx_v[pl.ds(i * 16, 16)]
            ck, ci = plsc.sort_key_val(ck, ci)
            return _kv_merge32_low(k0, v0, k1, v1,
                                   ck, ci, jnp.full((16,), _INF), ci)

        carry = (jnp.full((16,), _INF), _iota16(),
                 jnp.full((16,), _INF), _iota16())
        for q in range(4):
            carry = lax.fori_loop(q * (QCAP // 16), (offs[q] >> 4) + 1,
                                  _sel, carry)
        k0, v0, k1, v1 = carry
        vout_v[pl.ds(r * 32, 16)] = k0
        vout_v[pl.ds(r * 32 + 16, 16)] = k1
        iout_v[pl.ds(r * 32, 16)] = v0
        iout_v[pl.ds(r * 32 + 16, 16)] = v1
        for d in range(DIM):
            pout_v[pl.ds((d * ROWS_SEL + r) * 32, 16)] = \
                plsc.load_gather(ptab_v, [v0 + d * T])
            pout_v[pl.ds((d * ROWS_SEL + r) * 32 + 16, 16)] = \
                plsc.load_gather(ptab_v, [v1 + d * T])
        nxt = r + 2

        @pl.when(nxt < ROWS_SEL)
        def _():
            _issue(nxt, b)

    @pl.loop(0, ROWS_SEL, step=2)
    def _rows(r):
        for b in range(2):
            _row(r + b, b)

    pltpu.sync_copy(vout_v, val_hbm.at[pl.ds(rbase * 32, ROWS_SEL * 32)])
    pltpu.sync_copy(iout_v, idx_hbm.at[pl.ds(rbase * 32, ROWS_SEL * 32)])
    for d in range(DIM):
        pltpu.sync_copy(pout_v.at[pl.ds(d * ROWS_SEL * 32, ROWS_SEL * 32)],
                        pool_hbm.at[pl.ds((d * R + rbase) * 32, ROWS_SEL * 32)])


@functools.partial(
    pl.kernel,
    mesh=plsc.VectorSubcoreMesh(core_axis_name="c", subcore_axis_name="s"),
    out_type=[jax.ShapeDtypeStruct((R * 32,), jnp.float32),
              jax.ShapeDtypeStruct((R * 32,), jnp.int32),
              jax.ShapeDtypeStruct((DIM * R * 32,), jnp.float32)],
    compiler_params=pltpu.CompilerParams(needs_layout_passes=False),
    scratch_types=[
        pltpu.VMEM((ROWS_SEL, NCM), jnp.float32),
        pltpu.VMEM((2, T), jnp.float32),
        pltpu.VMEM((CAND + 128,), jnp.float32),
        pltpu.VMEM((CAND + 128,), jnp.int32),
        pltpu.VMEM((ROWS_SEL * 32,), jnp.float32),
        pltpu.VMEM((ROWS_SEL * 32,), jnp.int32),
        pltpu.VMEM((DIM * T,), jnp.float32),
        pltpu.VMEM((DIM * ROWS_SEL * 32,), jnp.float32),
        pltpu.SemaphoreType.DMA,
        pltpu.SemaphoreType.DMA,
    ],
)
def _sel_sc(d2_hbm, cm_hbm, pT_hbm, val_hbm, idx_hbm, pool_hbm, *scratch):
    _sel_body(d2_hbm, cm_hbm, pT_hbm, val_hbm, idx_hbm, pool_hbm, *scratch)


SROWS = 128


def _scores_body(ctxT_ref, ws1T_ref, bs1_ref, ws2_ref, s_ref):
    ctx_t = ctxT_ref[...]
    for i in range(NB):
        Et = jnp.dot(ws1T_ref[i], ctx_t, preferred_element_type=jnp.float32)
        Et = jnp.maximum(Et + bs1_ref[i][:, None], 0.0)
        s_ref[i] = jnp.dot(ws2_ref[i][None, :], Et,
                           preferred_element_type=jnp.float32)[0]


def _scores_call(ctxT, ws1T, bs1, ws2):
    grid = (R // SROWS,)
    return pl.pallas_call(
        _scores_body,
        grid=grid,
        in_specs=[pl.BlockSpec((CTX, SROWS * K), lambda i: (0, i)),
                  pl.BlockSpec((NB, EK, CTX), lambda i: (0, 0, 0)),
                  pl.BlockSpec((NB, EK), lambda i: (0, 0)),
                  pl.BlockSpec((NB, EK), lambda i: (0, 0))],
        out_specs=pl.BlockSpec((NB, SROWS * K), lambda i: (0, i)),
        out_shape=jax.ShapeDtypeStruct((NB, R * K), jnp.float32),
    )(ctxT, ws1T, bs1, ws2)


def _softmax_body(s_ref, a_ref):
    sm = s_ref[0]
    sm = sm - jnp.max(sm, axis=1, keepdims=True)
    ex = jnp.exp(sm)
    a_ref[0] = ex / jnp.sum(ex, axis=1, keepdims=True)


def _softmax_call(s3):
    grid = (NB * (R // 1024),)
    nrt = R // 1024
    return pl.pallas_call(
        _softmax_body,
        grid=grid,
        in_specs=[pl.BlockSpec((1, 1024, K), lambda i: (i // nrt, i % nrt, 0))],
        out_specs=pl.BlockSpec((1, 1024, K), lambda i: (i // nrt, i % nrt, 0)),
        out_shape=jax.ShapeDtypeStruct((NB, R, K), jnp.float32),
    )(s3)


def _affine_body(wt_ref, b_ref, x_ref, out_ref):
    out_ref[...] = (jnp.dot(wt_ref[...], x_ref[...],
                            preferred_element_type=jnp.float32) + b_ref[...])


def _affine_call(wt, b, x):
    din, dout = wt.shape[1], wt.shape[0]
    grid = (R // ROWS_BLK,)
    return pl.pallas_call(
        _affine_body,
        grid=grid,
        in_specs=[pl.BlockSpec((dout, din), lambda i: (0, 0)),
                  pl.BlockSpec((dout, 1), lambda i: (0, 0)),
                  pl.BlockSpec((din, ROWS_BLK), lambda i: (0, i))],
        out_specs=pl.BlockSpec((dout, ROWS_BLK), lambda i: (0, i)),
        out_shape=jax.ShapeDtypeStruct((dout, R), jnp.float32),
    )(wt, b.reshape(dout, 1), x)


def _affine_t_body(wt_ref, b_ref, x_ref, out_ref):
    out_ref[...] = (jnp.dot(wt_ref[...], x_ref[...],
                            preferred_element_type=jnp.float32)
                    + b_ref[...]).T


def _affine_t_call(wt, b, x):
    din, dout = wt.shape[1], wt.shape[0]
    grid = (R // ROWS_BLK,)
    return pl.pallas_call(
        _affine_t_body,
        grid=grid,
        in_specs=[pl.BlockSpec((dout, din), lambda i: (0, 0)),
                  pl.BlockSpec((dout, 1), lambda i: (0, 0)),
                  pl.BlockSpec((din, ROWS_BLK), lambda i: (0, i))],
        out_specs=pl.BlockSpec((ROWS_BLK, dout), lambda i: (i, 0)),
        out_shape=jax.ShapeDtypeStruct((R, dout), jnp.float32),
    )(wt, b.reshape(dout, 1), x)


def _block_body(net_ref, wpool_ref, last_ref,
                WvT_ref, bv_ref, fc0aT_ref, fc0bT_ref, fc0bias_ref,
                fc1T_ref, fc1b_ref, scaT_ref, scbT_ref,
                out_ref):
    net = net_ref[...]
    att = jnp.dot(WvT_ref[...], wpool_ref[...],
                  preferred_element_type=jnp.float32) + bv_ref[...]
    rnet = jnp.maximum(net, 0.0)
    ratt = jnp.maximum(att, 0.0)
    h = (jnp.dot(fc0aT_ref[...], rnet, preferred_element_type=jnp.float32)
         + jnp.dot(fc0bT_ref[...], ratt, preferred_element_type=jnp.float32)
         + fc0bias_ref[...])
    dx = jnp.dot(fc1T_ref[...], jnp.maximum(h, 0.0),
                 preferred_element_type=jnp.float32) + fc1b_ref[...]
    sc = (jnp.dot(scaT_ref[...], net, preferred_element_type=jnp.float32)
          + jnp.dot(scbT_ref[...], att, preferred_element_type=jnp.float32))
    out_ref[...] = sc + dx + last_ref[...]


def _run_block(netT, wpoolT, lastT, WvT, bv, fc0T, fc0bias, fc1T, fc1b, scT):
    grid = (R // ROWS_BLK,)
    row_spec = pl.BlockSpec((HID, ROWS_BLK), lambda i: (0, i))
    w_spec = pl.BlockSpec((HID, HID), lambda i: (0, 0))
    b_spec = pl.BlockSpec((HID, 1), lambda i: (0, 0))
    return pl.pallas_call(
        _block_body,
        grid=grid,
        in_specs=[row_spec, row_spec, row_spec,
                  w_spec, b_spec, w_spec, w_spec, b_spec,
                  w_spec, b_spec, w_spec, w_spec],
        out_specs=row_spec,
        out_shape=jax.ShapeDtypeStruct((HID, R), jnp.float32),
    )(netT, wpoolT, lastT,
      WvT, bv.reshape(HID, 1), fc0T[:, :HID], fc0T[:, HID:],
      fc0bias.reshape(HID, 1), fc1T, fc1b.reshape(HID, 1),
      scT[:, :HID], scT[:, HID:])


def kernel(p, fc_pos_W, fc_pos_b, fc0_W, fc0_b, fc1_W, fc1_b, sc_W,
           att_Ws1, att_bs1, att_Ws2, att_bs2, att_Wv, att_bv, fcc_W, fcc_b):
    pT = jnp.transpose(p, (0, 2, 1))
    d2full, cmins = _d2_call(p, pT)
    selval, selidx, poolT = _sel_sc(d2full, cmins, pT.reshape(B * DIM * T))
    selval = selval.reshape(R, 32)
    selidx = selidx.reshape(R, 32)
    poolT = poolT.reshape(DIM, R, 32)
    idx = selidx[:, :K].reshape(B, T, K)
    dis = jnp.sqrt(jnp.maximum(selval[:, :K], 1e-12)).reshape(B, T, K)

    prep = jnp.broadcast_to(p.reshape(R, DIM).T[:, :, None], (DIM, R, K))
    ctxT = jnp.concatenate(
        [dis.reshape(1, R * K), poolT[:, :, :K].reshape(DIM, R * K),
         prep.reshape(DIM, R * K)], axis=0)
    s = _scores_call(ctxT, jnp.transpose(att_Ws1, (0, 2, 1)),
                     att_bs1, att_Ws2[:, :, 0])
    a3 = _softmax_call(s.reshape(NB, R, K))
    wflat = a3.reshape(NB, R * K)

    idxT = idx.reshape(R, K).T.astype(jnp.int32)

    netT = _affine_call(fc_pos_W.T, fc_pos_b, p.reshape(R, DIM).T)
    lastT = jnp.zeros_like(netT)
    for i in range(NB):
        wpoolT = _wpool_sc(netT, idxT, wflat[i])
        netT = _run_block(netT, wpoolT, lastT,
                          att_Wv[i].T, att_bv[i],
                          fc0_W[i].T, fc0_b[i], fc1_W[i].T, fc1_b[i],
                          sc_W[i].T)
        lastT = netT
    c = _affine_t_call(fcc_W.T, fcc_b, netT)
    return c.reshape(B, T, C_DIM)
